# Initial kernel scaffold; baseline (speedup 1.0000x reference)
import jax
import jax.numpy as jnp
from jax.experimental import pallas as pl


def kernel(old_token_tensor, new_token_tensor, action_tensor, nl_tensor, code_table, action_table, nl_table):
    raise NotImplementedError("write your pallas kernel here")


# speedup vs baseline: 4.3230x; 4.3230x over previous
import functools

import jax
import jax.numpy as jnp
from jax import lax
from jax.experimental import pallas as pl
from jax.experimental.pallas import tpu as pltpu
from jax.experimental.pallas import tpu_sc as plsc

NC, NS = 2, 16
NW = NC * NS
LANE = 128
ROWS_PER_CHUNK = 8
CHUNK = LANE * ROWS_PER_CHUNK


def _gather_one(table, idx, out, idx_v, rows_v, sem, row_base, n_chunks):
    def body(c, carry):
        r0 = row_base + c * ROWS_PER_CHUNK
        pltpu.sync_copy(idx.at[pl.ds(r0, ROWS_PER_CHUNK)], idx_v)
        cps = [
            pltpu.async_copy(
                table.at[idx_v.at[j]],
                rows_v.at[pl.ds(j * LANE, LANE)],
                sem,
            )
            for j in range(ROWS_PER_CHUNK)
        ]
        for cp in cps:
            cp.wait()
        pltpu.sync_copy(rows_v, out.at[pl.ds(r0 * LANE, CHUNK)])
        return carry

    lax.fori_loop(0, n_chunks, body, 0)


def _sc_body(old_i, new_i, act_i, nl_i, code_t, act_t, nl_t,
             old_o, new_o, act_o, nl_o, idx_v, rows_v, sem):
    wid = lax.axis_index("s") * NC + lax.axis_index("c")
    rows_total = old_i.shape[0]
    rows_per_w = rows_total // NW
    n_chunks = rows_per_w // ROWS_PER_CHUNK
    row_base = wid * rows_per_w
    _gather_one(code_t, old_i, old_o, idx_v, rows_v, sem, row_base, n_chunks)
    _gather_one(code_t, new_i, new_o, idx_v, rows_v, sem, row_base, n_chunks)
    _gather_one(act_t, act_i, act_o, idx_v, rows_v, sem, row_base, n_chunks)
    _gather_one(nl_t, nl_i, nl_o, idx_v, rows_v, sem, row_base, n_chunks)


def kernel(old_token_tensor, new_token_tensor, action_tensor, nl_tensor,
           code_table, action_table, nl_table):
    B, L = old_token_tensor.shape
    E = code_table.shape[1]
    n = B * L
    ids = [
        t.reshape(n // LANE, LANE).astype(jnp.int32)
        for t in (old_token_tensor, new_token_tensor, action_tensor, nl_tensor)
    ]
    out_t = [jax.ShapeDtypeStruct((n, E), jnp.float32)] * 4
    mesh = plsc.VectorSubcoreMesh(
        core_axis_name="c", subcore_axis_name="s",
        num_cores=NC, num_subcores=NS,
    )
    run = pl.kernel(
        _sc_body,
        out_type=out_t,
        mesh=mesh,
        scratch_types=[
            pltpu.VMEM((ROWS_PER_CHUNK, LANE), jnp.int32),
            pltpu.VMEM((CHUNK, E), jnp.float32),
            pltpu.SemaphoreType.DMA,
        ],
        compiler_params=pltpu.CompilerParams(use_tc_tiling_on_sc=False),
    )
    outs = run(*ids, code_table, action_table, nl_table)
    return tuple(o.reshape(B, L, E) for o in outs)


# speedup vs baseline: 4.4563x; 1.0308x over previous
import jax
import jax.numpy as jnp
from jax import lax
from jax.experimental import pallas as pl
from jax.experimental.pallas import tpu as pltpu
from jax.experimental.pallas import tpu_sc as plsc

NC, NS = 2, 16
NW = NC * NS
LANE = 128
RPC = 4
CHUNK = LANE * RPC
NBUF = 2


def _gather_one(table, idx, out, idx_v, bufs, gsems, ssems, row_base,
                rows_per_w):
    n_chunks = rows_per_w // RPC
    E = table.shape[1]

    pltpu.sync_copy(idx.at[pl.ds(row_base, rows_per_w)], idx_v)

    def fire_gathers(c, b):
        for j in range(RPC):
            pltpu.async_copy(
                table.at[idx_v.at[c * RPC + j]],
                bufs[b].at[pl.ds(j * LANE, LANE)],
                gsems[b],
            )

    def wait_gathers(b):
        pltpu.make_async_copy(out.at[pl.ds(0, CHUNK)], bufs[b],
                              gsems[b]).wait()

    def out_slice(c):
        return out.at[pl.ds((row_base + c * RPC) * LANE, CHUNK)]

    def wait_store(b):
        pltpu.make_async_copy(bufs[b], out_slice(0), ssems[b]).wait()

    for b in range(NBUF):
        fire_gathers(b, b)

    def body(i, carry):
        for b in range(NBUF):
            c = i * NBUF + b
            wait_gathers(b)
            pltpu.async_copy(bufs[b], out_slice(c), ssems[b])

            @pl.when(c + NBUF < n_chunks)
            def _():
                wait_store(b)
                fire_gathers(c + NBUF, b)

        return carry

    lax.fori_loop(0, n_chunks // NBUF, body, 0)
    for b in range(NBUF):
        wait_store(b)


def _sc_body(old_i, new_i, act_i, nl_i, code_t, act_t, nl_t,
             old_o, new_o, act_o, nl_o, idx_v, buf0, buf1,
             gsem0, gsem1, ssem0, ssem1):
    wid = lax.axis_index("s") * NC + lax.axis_index("c")
    rows_total = old_i.shape[0]
    rows_per_w = rows_total // NW
    row_base = wid * rows_per_w
    bufs = (buf0, buf1)
    gsems = (gsem0, gsem1)
    ssems = (ssem0, ssem1)
    for table, idx, out in ((code_t, old_i, old_o), (code_t, new_i, new_o),
                            (act_t, act_i, act_o), (nl_t, nl_i, nl_o)):
        _gather_one(table, idx, out, idx_v, bufs, gsems, ssems, row_base,
                    rows_per_w)


def kernel(old_token_tensor, new_token_tensor, action_tensor, nl_tensor,
           code_table, action_table, nl_table):
    B, L = old_token_tensor.shape
    E = code_table.shape[1]
    n = B * L
    rows_per_w = (n // LANE) // NW
    ids = [
        t.reshape(n // LANE, LANE).astype(jnp.int32)
        for t in (old_token_tensor, new_token_tensor, action_tensor, nl_tensor)
    ]
    out_t = [jax.ShapeDtypeStruct((n, E), jnp.float32)] * 4
    mesh = plsc.VectorSubcoreMesh(
        core_axis_name="c", subcore_axis_name="s",
        num_cores=NC, num_subcores=NS,
    )
    run = pl.kernel(
        _sc_body,
        out_type=out_t,
        mesh=mesh,
        scratch_types=[
            pltpu.VMEM((rows_per_w, LANE), jnp.int32),
            pltpu.VMEM((CHUNK, E), jnp.float32),
            pltpu.VMEM((CHUNK, E), jnp.float32),
            pltpu.SemaphoreType.DMA,
            pltpu.SemaphoreType.DMA,
            pltpu.SemaphoreType.DMA,
            pltpu.SemaphoreType.DMA,
        ],
        compiler_params=pltpu.CompilerParams(use_tc_tiling_on_sc=False),
    )
    outs = run(*ids, code_table, action_table, nl_table)
    return tuple(o.reshape(B, L, E) for o in outs)


# speedup vs baseline: 4.4648x; 1.0019x over previous
import jax
import jax.numpy as jnp
from jax import lax
from jax.experimental import pallas as pl
from jax.experimental.pallas import tpu as pltpu
from jax.experimental.pallas import tpu_sc as plsc

NC, NS = 2, 16
NW = NC * NS
NB = 2
NBUF = 2
SEGS = ((0, 128), (128, 72))


def _gather_one(table, idx, out, idx_v, bufs, gsems, ssems, b_base, b_per_w):
    n_chunks = b_per_w // NB

    pltpu.sync_copy(idx.at[pl.ds(b_base, b_per_w)], idx_v)

    def fire_gathers(c, b):
        for r in range(NB):
            for off, ln in SEGS:
                pltpu.async_copy(
                    table.at[idx_v.at[c * NB + r, pl.ds(off, ln)]],
                    bufs[b].at[r, pl.ds(off, ln)],
                    gsems[b],
                )

    def wait_gathers(b):
        pltpu.make_async_copy(out.at[pl.ds(0, NB)], bufs[b], gsems[b]).wait()

    def wait_store(b):
        pltpu.make_async_copy(bufs[b], out.at[pl.ds(0, NB)], ssems[b]).wait()

    for b in range(NBUF):
        fire_gathers(b, b)

    def body(i, carry):
        for b in range(NBUF):
            c = i * NBUF + b
            wait_gathers(b)
            pltpu.async_copy(bufs[b], out.at[pl.ds(b_base + c * NB, NB)],
                             ssems[b])

            @pl.when(c + NBUF < n_chunks)
            def _():
                wait_store(b)
                fire_gathers(c + NBUF, b)

        return carry

    lax.fori_loop(0, n_chunks // NBUF, body, 0)
    for b in range(NBUF):
        wait_store(b)


def _sc_body(old_i, new_i, act_i, nl_i, code_t, act_t, nl_t,
             old_o, new_o, act_o, nl_o, idx_v, buf0, buf1,
             gsem0, gsem1, ssem0, ssem1):
    wid = lax.axis_index("s") * NC + lax.axis_index("c")
    B = old_i.shape[0]
    b_per_w = B // NW
    b_base = wid * b_per_w
    bufs = (buf0, buf1)
    gsems = (gsem0, gsem1)
    ssems = (ssem0, ssem1)
    for table, idx, out in ((code_t, old_i, old_o), (code_t, new_i, new_o),
                            (act_t, act_i, act_o), (nl_t, nl_i, nl_o)):
        _gather_one(table, idx, out, idx_v, bufs, gsems, ssems, b_base,
                    b_per_w)


def kernel(old_token_tensor, new_token_tensor, action_tensor, nl_tensor,
           code_table, action_table, nl_table):
    B, L = old_token_tensor.shape
    E = code_table.shape[1]
    b_per_w = B // NW
    ids = [
        t.astype(jnp.int32)
        for t in (old_token_tensor, new_token_tensor, action_tensor, nl_tensor)
    ]
    out_t = [jax.ShapeDtypeStruct((B, L, E), jnp.float32)] * 4
    mesh = plsc.VectorSubcoreMesh(
        core_axis_name="c", subcore_axis_name="s",
        num_cores=NC, num_subcores=NS,
    )
    run = pl.kernel(
        _sc_body,
        out_type=out_t,
        mesh=mesh,
        scratch_types=[
            pltpu.VMEM((b_per_w, L), jnp.int32),
            pltpu.VMEM((NB, L, E), jnp.float32),
            pltpu.VMEM((NB, L, E), jnp.float32),
            pltpu.SemaphoreType.DMA,
            pltpu.SemaphoreType.DMA,
            pltpu.SemaphoreType.DMA,
            pltpu.SemaphoreType.DMA,
        ],
        compiler_params=pltpu.CompilerParams(use_tc_tiling_on_sc=False),
    )
    return tuple(run(*ids, code_table, action_table, nl_table))


# speedup vs baseline: 4.8928x; 1.0959x over previous
import jax
import jax.numpy as jnp
from jax import lax
from jax.experimental import pallas as pl
from jax.experimental.pallas import tpu as pltpu
from jax.experimental.pallas import tpu_sc as plsc

NC, NS = 2, 16
NW = NC * NS
NB = 2
NBUF = 2
SEGS = ((0, 128), (128, 72))


def _gather_one(table, idx, out, idx_v, bufs, gsems, ssems, b_base, b_per_w):
    n_chunks = b_per_w // NB

    pltpu.sync_copy(idx.at[pl.ds(b_base, b_per_w)], idx_v)

    def fire_gathers(c, b):
        for r in range(NB):
            for off, ln in SEGS:
                pltpu.async_copy(
                    table.at[idx_v.at[c * NB + r, pl.ds(off, ln)]],
                    bufs[b].at[r, pl.ds(off, ln)],
                    gsems[b],
                )

    def wait_gathers(b):
        pltpu.make_async_copy(out.at[pl.ds(0, NB)], bufs[b], gsems[b]).wait()

    def wait_store(b):
        pltpu.make_async_copy(bufs[b], out.at[pl.ds(0, NB)], ssems[b]).wait()

    for b in range(NBUF):
        fire_gathers(b, b)

    def body(i, carry):
        for b in range(NBUF):
            c = i * NBUF + b
            wait_gathers(b)
            pltpu.async_copy(bufs[b], out.at[pl.ds(b_base + c * NB, NB)],
                             ssems[b])

            @pl.when(c + NBUF < n_chunks)
            def _():
                wait_store(b)
                fire_gathers(c + NBUF, b)

        return carry

    lax.fori_loop(0, n_chunks // NBUF, body, 0)
    for b in range(NBUF):
        wait_store(b)


def _sc_body(idx, table, out, idx_v, buf0, buf1,
             gsem0, gsem1, ssem0, ssem1):
    wid = lax.axis_index("s") * NC + lax.axis_index("c")
    B = idx.shape[0]
    b_per_w = B // NW
    b_base = wid * b_per_w
    _gather_one(table, idx, out, idx_v, (buf0, buf1), (gsem0, gsem1),
                (ssem0, ssem1), b_base, b_per_w)


def kernel(old_token_tensor, new_token_tensor, action_tensor, nl_tensor,
           code_table, action_table, nl_table):
    B, L = old_token_tensor.shape
    E = code_table.shape[1]
    b_per_w = B // NW
    ids = [
        t.astype(jnp.int32)
        for t in (old_token_tensor, new_token_tensor, action_tensor, nl_tensor)
    ]
    mesh = plsc.VectorSubcoreMesh(
        core_axis_name="c", subcore_axis_name="s",
        num_cores=NC, num_subcores=NS,
    )
    run = pl.kernel(
        _sc_body,
        out_type=jax.ShapeDtypeStruct((B, L, E), jnp.float32),
        mesh=mesh,
        scratch_types=[
            pltpu.VMEM((b_per_w, L), jnp.int32),
            pltpu.VMEM((NB, L, E), jnp.float32),
            pltpu.VMEM((NB, L, E), jnp.float32),
            pltpu.SemaphoreType.DMA,
            pltpu.SemaphoreType.DMA,
            pltpu.SemaphoreType.DMA,
            pltpu.SemaphoreType.DMA,
        ],
        compiler_params=pltpu.CompilerParams(use_tc_tiling_on_sc=False),
    )
    tables = (code_table, code_table, action_table, nl_table)
    return tuple(run(i, t) for i, t in zip(ids, tables))
